# Initial kernel scaffold; baseline (speedup 1.0000x reference)
import jax
import jax.numpy as jnp
from jax.experimental import pallas as pl


def kernel(logits):
    raise NotImplementedError("write your pallas kernel here")


# speedup vs baseline: 14.8335x; 14.8335x over previous
import functools

import jax
import jax.numpy as jnp
import numpy as np
from jax import lax
from jax.experimental import pallas as pl
from jax.experimental.pallas import tpu as pltpu
from jax.experimental.pallas import tpu_sc as plsc

TOPK = 50
L = 16
CAP = 256
BUFCAP = CAP + L
RESEL_AT = CAP - L
NBUF = BUFCAP // L
NOUT = CAP // L


def _popcount(mask):
    ones = jnp.full((L,), jnp.int32(1))
    zeros = jnp.full((L,), jnp.int32(0))
    return jnp.sum(jnp.where(mask, ones, zeros))


def _sc_collect(logits):
    B, V = logits.shape
    nvreg = V // L
    info = plsc.get_sparse_core_info()
    nw = info.num_cores * info.num_subcores
    rows_per_w = B // nw
    mesh = plsc.VectorSubcoreMesh(core_axis_name="c", subcore_axis_name="s")

    @functools.partial(
        pl.kernel,
        out_type=(jax.ShapeDtypeStruct((B, CAP), jnp.float32),
                  jax.ShapeDtypeStruct((B, CAP), jnp.int32)),
        mesh=mesh,
        scratch_types=[
            pltpu.VMEM((V,), jnp.float32),
            pltpu.VMEM((BUFCAP,), jnp.float32),
            pltpu.VMEM((BUFCAP,), jnp.int32),
            pltpu.SMEM((1,), jnp.int32),
            pltpu.SMEM((1,), jnp.float32),
        ],
        compiler_params=pltpu.CompilerParams(needs_layout_passes=False),
    )
    def body(logits_hbm, vals_hbm, idx_hbm, row_v, cv, ci, smc, smt):
        cid = lax.axis_index("c")
        sid = lax.axis_index("s")
        wid = sid * info.num_cores + cid
        lane = lax.iota(jnp.int32, L)

        def ordkey(v):
            sb = plsc.bitcast(v, jnp.int32)
            neg = plsc.bitcast(sb >> 31, jnp.uint32)
            return plsc.bitcast(sb, jnp.uint32) ^ (neg | jnp.uint32(0x80000000))

        def reselect():
            cc = smc[0]

            def count_ge(mid):
                midv = jnp.full((L,), mid)
                total = jnp.int32(0)
                for j in range(NBUF):
                    k = ordkey(cv[pl.ds(j * L, L)])
                    ok = (k >= midv) & ((lane + j * L) < cc)
                    total = total + _popcount(ok)
                return total

            def bs_step(_, lh):
                lo, hi = lh
                mid = lo + ((hi - lo + jnp.uint32(1)) >> jnp.uint32(1))
                big = count_ge(mid) >= TOPK
                return (jnp.where(big, mid, lo),
                        jnp.where(big, hi, mid - jnp.uint32(1)))

            lo, _ = lax.fori_loop(
                0, 32, bs_step, (jnp.uint32(0), jnp.uint32(0xFFFFFFFE)))
            bits = jnp.where(lo >= jnp.uint32(0x80000000),
                             lo ^ jnp.uint32(0x80000000), ~lo)
            t_new = lax.bitcast_convert_type(bits, jnp.float32)
            smt[0] = t_new
            tv = jnp.full((L,), t_new)

            def comp_step(j, c2):
                v = cv[pl.ds(j * L, L)]
                ix = ci[pl.ds(j * L, L)]
                keep = (v >= tv) & ((lane + j * L) < cc)
                plsc.store_compressed(cv.at[pl.ds(c2, L)], v, mask=keep)
                plsc.store_compressed(ci.at[pl.ds(c2, L)], ix, mask=keep)
                return c2 + _popcount(keep)

            smc[0] = lax.fori_loop(0, NBUF, comp_step, jnp.int32(0))

        def scan_step(i, carry):
            c = smc[0]
            t = smt[0]
            v = row_v[pl.ds(i * L, L)]
            mask = v >= jnp.full((L,), t)
            n = _popcount(mask)

            @pl.when(n > 0)
            def _():
                plsc.store_compressed(cv.at[pl.ds(c, L)], v, mask=mask)
                plsc.store_compressed(ci.at[pl.ds(c, L)], lane + i * L,
                                      mask=mask)
                smc[0] = c + n

                @pl.when(c + n > RESEL_AT)
                def _():
                    reselect()

            return carry

        for rr in range(rows_per_w):
            r = wid * rows_per_w + rr
            pltpu.sync_copy(logits_hbm.at[r], row_v)
            smc[0] = jnp.int32(0)
            smt[0] = jnp.float32(-jnp.inf)
            lax.fori_loop(0, nvreg, scan_step, 0)
            cc = smc[0]
            for j in range(NOUT):
                lm = (lane + j * L) < cc
                cv[pl.ds(j * L, L)] = jnp.where(
                    lm, cv[pl.ds(j * L, L)], jnp.float32(-jnp.inf))
                ci[pl.ds(j * L, L)] = jnp.where(
                    lm, ci[pl.ds(j * L, L)], jnp.int32(0))
            pltpu.sync_copy(cv.at[pl.ds(0, CAP)], vals_hbm.at[r])
            pltpu.sync_copy(ci.at[pl.ds(0, CAP)], idx_hbm.at[r])

    return body(logits)


def _tail_body(kd_ref, vals_ref, idx_ref, out_ref, *, vocab):
    v = vals_ref[...]
    ix = idx_ref[...]
    B = v.shape[0]
    k0 = kd_ref[0]
    k1 = kd_ref[1]

    sb = lax.bitcast_convert_type(v, jnp.int32)
    neg = lax.bitcast_convert_type(sb >> 31, jnp.uint32)
    keys = lax.bitcast_convert_type(sb, jnp.uint32) ^ (
        neg | jnp.uint32(0x80000000))

    lo = jnp.zeros((B, 1), jnp.uint32)
    hi = jnp.full((B, 1), jnp.uint32(0xFFFFFFFE))
    for _ in range(32):
        mid = lo + ((hi - lo + jnp.uint32(1)) >> jnp.uint32(1))
        cnt = jnp.sum((keys >= mid).astype(jnp.int32), axis=1, keepdims=True)
        big = cnt >= TOPK
        lo = jnp.where(big, mid, lo)
        hi = jnp.where(big, hi, mid - jnp.uint32(1))
    mk = lo
    mv_bits = jnp.where(mk >= jnp.uint32(0x80000000),
                        mk ^ jnp.uint32(0x80000000), ~mk)
    min_val = lax.bitcast_convert_type(mv_bits, jnp.float32)

    m = jnp.max(v, axis=1, keepdims=True)
    sup = v >= min_val
    e = jnp.where(sup, jnp.exp(v - m), jnp.float32(0.0))
    denom = jnp.sum(e, axis=1, keepdims=True)
    lp = jnp.log(e / denom + jnp.float32(1e-30))

    row = lax.broadcasted_iota(jnp.int32, v.shape, 0)
    lin = (row * vocab + ix).astype(jnp.uint32)
    x0 = jnp.zeros_like(lin)
    x1 = lin
    ks2 = k0 ^ k1 ^ jnp.uint32(0x1BD11BDA)

    def rotl(x, r):
        return (x << jnp.uint32(r)) | (x >> jnp.uint32(32 - r))

    x0 = x0 + k0
    x1 = x1 + k1
    rots = ((13, 15, 26, 6), (17, 29, 16, 24))
    kadd = ((k1, ks2 + jnp.uint32(1)), (ks2, k0 + jnp.uint32(2)),
            (k0, k1 + jnp.uint32(3)), (k1, ks2 + jnp.uint32(4)),
            (ks2, k0 + jnp.uint32(5)))
    for i in range(5):
        for r in rots[i % 2]:
            x0 = x0 + x1
            x1 = rotl(x1, r)
            x1 = x1 ^ x0
        x0 = x0 + kadd[i][0]
        x1 = x1 + kadd[i][1]
    bits = x0 ^ x1

    fb = (bits >> jnp.uint32(9)) | jnp.uint32(0x3F800000)
    u = lax.bitcast_convert_type(fb, jnp.float32) - jnp.float32(1.0)
    tiny = np.float32(np.finfo(np.float32).tiny)
    u2 = jnp.maximum(tiny, u * (np.float32(1.0) - tiny) + tiny)
    g = -jnp.log(-jnp.log(u2))

    score = jnp.where(sup, lp + g, -jnp.inf)
    best = jnp.max(score, axis=1, keepdims=True)
    samp = jnp.min(jnp.where(score == best, ix, jnp.int32(0x7FFFFFFF)),
                   axis=1, keepdims=True)
    out_ref[...] = samp


def _tail(key_data, vals, idx, vocab):
    B = vals.shape[0]
    return pl.pallas_call(
        functools.partial(_tail_body, vocab=vocab),
        out_shape=jax.ShapeDtypeStruct((B, 1), jnp.int32),
        in_specs=[
            pl.BlockSpec(memory_space=pltpu.SMEM),
            pl.BlockSpec(memory_space=pltpu.VMEM),
            pl.BlockSpec(memory_space=pltpu.VMEM),
        ],
        out_specs=pl.BlockSpec(memory_space=pltpu.VMEM),
    )(key_data, vals, idx)


def kernel(logits):
    if logits.ndim == 3:
        logits = jnp.squeeze(logits, axis=1)
    B, V = logits.shape
    vals, idx = _sc_collect(logits)
    kd = jax.random.key_data(
        jax.random.fold_in(jax.random.key(0), 1)).astype(jnp.uint32)
    return _tail(kd, vals, idx, V)


# speedup vs baseline: 29.6761x; 2.0006x over previous
import functools

import jax
import jax.numpy as jnp
import numpy as np
from jax import lax
from jax.experimental import pallas as pl
from jax.experimental.pallas import tpu as pltpu
from jax.experimental.pallas import tpu_sc as plsc

TOPK = 50
L = 16
CAP = 256
BUFCAP = CAP + L
RESEL_AT = CAP - L
NBUF = BUFCAP // L
NOUT = CAP // L
BOOT = 10
G = 8


def _popcount(mask):
    pc = plsc.all_reduce_population_count(mask)
    return lax.squeeze(lax.slice(pc, (0,), (1,)), (0,))


def _sc_collect(logits):
    B, V = logits.shape
    nvreg = V // L
    info = plsc.get_sparse_core_info()
    nw = info.num_cores * info.num_subcores
    rows_per_w = B // nw
    mesh = plsc.VectorSubcoreMesh(core_axis_name="c", subcore_axis_name="s")

    @functools.partial(
        pl.kernel,
        out_type=(jax.ShapeDtypeStruct((B, CAP), jnp.float32),
                  jax.ShapeDtypeStruct((B, CAP), jnp.int32)),
        mesh=mesh,
        scratch_types=[
            pltpu.VMEM((V,), jnp.float32),
            pltpu.VMEM((BUFCAP,), jnp.float32),
            pltpu.VMEM((BUFCAP,), jnp.int32),
            pltpu.SMEM((1,), jnp.int32),
            pltpu.SMEM((1,), jnp.float32),
        ],
        compiler_params=pltpu.CompilerParams(needs_layout_passes=False),
    )
    def body(logits_hbm, vals_hbm, idx_hbm, row_v, cv, ci, smc, smt):
        cid = lax.axis_index("c")
        sid = lax.axis_index("s")
        wid = sid * info.num_cores + cid
        lane = lax.iota(jnp.int32, L)

        def ordkey(v):
            sb = plsc.bitcast(v, jnp.int32)
            neg = plsc.bitcast(sb >> 31, jnp.uint32)
            return plsc.bitcast(sb, jnp.uint32) ^ (neg | jnp.uint32(0x80000000))

        def reselect():
            cc = smc[0]

            def count_ge(mid):
                midv = jnp.full((L,), mid)
                total = jnp.int32(0)
                for j in range(NBUF):
                    k = ordkey(cv[pl.ds(j * L, L)])
                    ok = (k >= midv) & ((lane + j * L) < cc)
                    total = total + _popcount(ok)
                return total

            def bs_step(_, lh):
                lo, hi = lh
                mid = lo + ((hi - lo + jnp.uint32(1)) >> jnp.uint32(1))
                big = count_ge(mid) >= TOPK
                return (jnp.where(big, mid, lo),
                        jnp.where(big, hi, mid - jnp.uint32(1)))

            lo, _ = lax.fori_loop(
                0, 32, bs_step, (jnp.uint32(0), jnp.uint32(0xFFFFFFFE)))
            bits = jnp.where(lo >= jnp.uint32(0x80000000),
                             lo ^ jnp.uint32(0x80000000), ~lo)
            t_new = lax.bitcast_convert_type(bits, jnp.float32)
            smt[0] = t_new
            tv = jnp.full((L,), t_new)

            def comp_step(j, c2):
                v = cv[pl.ds(j * L, L)]
                ix = ci[pl.ds(j * L, L)]
                keep = (v >= tv) & ((lane + j * L) < cc)
                plsc.store_compressed(cv.at[pl.ds(c2, L)], v, mask=keep)
                plsc.store_compressed(ci.at[pl.ds(c2, L)], ix, mask=keep)
                return c2 + _popcount(keep)

            smc[0] = lax.fori_loop(0, NBUF, comp_step, jnp.int32(0))

        def boot_step(i, carry):
            cv[pl.ds(i * L, L)] = row_v[pl.ds(i * L, L)]
            ci[pl.ds(i * L, L)] = lane + i * L
            return carry

        def group_step(gi, carry):
            t = smt[0]
            base = (BOOT + gi * G) * L
            mx = row_v[pl.ds(base, L)]
            for k in range(1, G):
                mx = jnp.maximum(mx, row_v[pl.ds(base + k * L, L)])
            gmask = mx >= jnp.full((L,), t)
            gn = _popcount(gmask)

            @pl.when(gn > 0)
            def _():
                tv = jnp.full((L,), smt[0])
                for k in range(G):
                    c = smc[0]
                    v = row_v[pl.ds(base + k * L, L)]
                    mask = v >= tv
                    n = _popcount(mask)

                    @pl.when(n > 0)
                    def _():
                        plsc.store_compressed(cv.at[pl.ds(c, L)], v,
                                              mask=mask)
                        plsc.store_compressed(ci.at[pl.ds(c, L)],
                                              lane + base + k * L, mask=mask)
                        smc[0] = c + n

                        @pl.when(c + n > RESEL_AT)
                        def _():
                            reselect()

            return carry

        for rr in range(rows_per_w):
            r = wid * rows_per_w + rr
            pltpu.sync_copy(logits_hbm.at[r], row_v)
            lax.fori_loop(0, BOOT, boot_step, 0)
            smc[0] = jnp.int32(BOOT * L)
            smt[0] = jnp.float32(-jnp.inf)
            reselect()
            lax.fori_loop(0, (nvreg - BOOT) // G, group_step, 0)
            cc = smc[0]
            for j in range(NOUT):
                lm = (lane + j * L) < cc
                cv[pl.ds(j * L, L)] = jnp.where(
                    lm, cv[pl.ds(j * L, L)], jnp.float32(-jnp.inf))
                ci[pl.ds(j * L, L)] = jnp.where(
                    lm, ci[pl.ds(j * L, L)], jnp.int32(0))
            pltpu.sync_copy(cv.at[pl.ds(0, CAP)], vals_hbm.at[r])
            pltpu.sync_copy(ci.at[pl.ds(0, CAP)], idx_hbm.at[r])

    return body(logits)


def _tail_body(kd_ref, vals_ref, idx_ref, out_ref, *, vocab):
    v = vals_ref[...]
    ix = idx_ref[...]
    B = v.shape[0]
    k0 = kd_ref[0]
    k1 = kd_ref[1]

    sb = lax.bitcast_convert_type(v, jnp.int32)
    neg = lax.bitcast_convert_type(sb >> 31, jnp.uint32)
    keys = lax.bitcast_convert_type(sb, jnp.uint32) ^ (
        neg | jnp.uint32(0x80000000))

    lo = jnp.zeros((B, 1), jnp.uint32)
    hi = jnp.full((B, 1), jnp.uint32(0xFFFFFFFE))
    for _ in range(32):
        mid = lo + ((hi - lo + jnp.uint32(1)) >> jnp.uint32(1))
        cnt = jnp.sum((keys >= mid).astype(jnp.int32), axis=1, keepdims=True)
        big = cnt >= TOPK
        lo = jnp.where(big, mid, lo)
        hi = jnp.where(big, hi, mid - jnp.uint32(1))
    mk = lo
    mv_bits = jnp.where(mk >= jnp.uint32(0x80000000),
                        mk ^ jnp.uint32(0x80000000), ~mk)
    min_val = lax.bitcast_convert_type(mv_bits, jnp.float32)

    m = jnp.max(v, axis=1, keepdims=True)
    sup = v >= min_val
    e = jnp.where(sup, jnp.exp(v - m), jnp.float32(0.0))
    denom = jnp.sum(e, axis=1, keepdims=True)
    lp = jnp.log(e / denom + jnp.float32(1e-30))

    row = lax.broadcasted_iota(jnp.int32, v.shape, 0)
    lin = (row * vocab + ix).astype(jnp.uint32)
    x0 = jnp.zeros_like(lin)
    x1 = lin
    ks2 = k0 ^ k1 ^ jnp.uint32(0x1BD11BDA)

    def rotl(x, r):
        return (x << jnp.uint32(r)) | (x >> jnp.uint32(32 - r))

    x0 = x0 + k0
    x1 = x1 + k1
    rots = ((13, 15, 26, 6), (17, 29, 16, 24))
    kadd = ((k1, ks2 + jnp.uint32(1)), (ks2, k0 + jnp.uint32(2)),
            (k0, k1 + jnp.uint32(3)), (k1, ks2 + jnp.uint32(4)),
            (ks2, k0 + jnp.uint32(5)))
    for i in range(5):
        for r in rots[i % 2]:
            x0 = x0 + x1
            x1 = rotl(x1, r)
            x1 = x1 ^ x0
        x0 = x0 + kadd[i][0]
        x1 = x1 + kadd[i][1]
    bits = x0 ^ x1

    fb = (bits >> jnp.uint32(9)) | jnp.uint32(0x3F800000)
    u = lax.bitcast_convert_type(fb, jnp.float32) - jnp.float32(1.0)
    tiny = np.float32(np.finfo(np.float32).tiny)
    u2 = jnp.maximum(tiny, u * (np.float32(1.0) - tiny) + tiny)
    g = -jnp.log(-jnp.log(u2))

    score = jnp.where(sup, lp + g, -jnp.inf)
    best = jnp.max(score, axis=1, keepdims=True)
    samp = jnp.min(jnp.where(score == best, ix, jnp.int32(0x7FFFFFFF)),
                   axis=1, keepdims=True)
    out_ref[...] = samp


def _tail(key_data, vals, idx, vocab):
    B = vals.shape[0]
    return pl.pallas_call(
        functools.partial(_tail_body, vocab=vocab),
        out_shape=jax.ShapeDtypeStruct((B, 1), jnp.int32),
        in_specs=[
            pl.BlockSpec(memory_space=pltpu.SMEM),
            pl.BlockSpec(memory_space=pltpu.VMEM),
            pl.BlockSpec(memory_space=pltpu.VMEM),
        ],
        out_specs=pl.BlockSpec(memory_space=pltpu.VMEM),
    )(key_data, vals, idx)


def kernel(logits):
    if logits.ndim == 3:
        logits = jnp.squeeze(logits, axis=1)
    B, V = logits.shape
    vals, idx = _sc_collect(logits)
    kd = jax.random.key_data(
        jax.random.fold_in(jax.random.key(0), 1)).astype(jnp.uint32)
    return _tail(kd, vals, idx, V)


# speedup vs baseline: 48.9312x; 1.6488x over previous
import functools

import jax
import jax.numpy as jnp
import numpy as np
from jax import lax
from jax.experimental import pallas as pl
from jax.experimental.pallas import tpu as pltpu
from jax.experimental.pallas import tpu_sc as plsc

TOPK = 50
L = 16
G = 16
BOOT = 74
BUFCAP = BOOT * L + L
NBUF = BUFCAP // L
RESEL_AT = BUFCAP - G * L
OUTCAP = 512
NEG_INF = float('-inf')


def _popcount(mask):
    pc = plsc.all_reduce_population_count(mask)
    return lax.squeeze(lax.slice(pc, (0,), (1,)), (0,))


def _sc_collect(logits):
    B, V = logits.shape
    nvreg = V // L
    ngroups = (nvreg - BOOT) // G
    assert BOOT + ngroups * G == nvreg
    info = plsc.get_sparse_core_info()
    nw = info.num_cores * info.num_subcores
    rows_per_w = B // nw
    mesh = plsc.VectorSubcoreMesh(core_axis_name="c", subcore_axis_name="s")

    @functools.partial(
        pl.kernel,
        out_type=(jax.ShapeDtypeStruct((B, OUTCAP), jnp.float32),
                  jax.ShapeDtypeStruct((B, OUTCAP), jnp.int32)),
        mesh=mesh,
        scratch_types=[
            pltpu.VMEM((V,), jnp.float32),
            pltpu.VMEM((BUFCAP,), jnp.float32),
            pltpu.VMEM((BUFCAP,), jnp.int32),
            pltpu.VMEM((L,), jnp.float32),
            pltpu.SMEM((1,), jnp.int32),
        ],
        compiler_params=pltpu.CompilerParams(needs_layout_passes=False),
    )
    def body(logits_hbm, vals_hbm, idx_hbm, row_v, cv, ci, tvr, smc):
        cid = lax.axis_index("c")
        sid = lax.axis_index("s")
        wid = sid * info.num_cores + cid
        lane = lax.iota(jnp.int32, L)

        def ordkey(v):
            sb = plsc.bitcast(v, jnp.int32)
            neg = plsc.bitcast(sb >> 31, jnp.uint32)
            return plsc.bitcast(sb, jnp.uint32) ^ (neg | jnp.uint32(0x80000000))

        def reselect(sv, iters):
            def bs_step(_, lh):
                lo, hi = lh
                one = jnp.full((L,), jnp.uint32(1))
                mid = lo + ((hi - lo + one) >> one)
                cnt = jnp.full((L,), jnp.int32(0))
                onei = jnp.full((L,), jnp.int32(1))
                zeroi = jnp.full((L,), jnp.int32(0))
                for j in range(sv):
                    k = ordkey(cv[pl.ds(j * L, L)])
                    cnt = cnt + jnp.where(k >= mid, onei, zeroi)
                big = cnt >= jnp.full((L,), jnp.int32(TOPK))
                return (jnp.where(big, mid, lo),
                        jnp.where(big, hi, mid - one))

            lo, _ = lax.fori_loop(
                0, iters, bs_step,
                (jnp.full((L,), jnp.uint32(0)),
                 jnp.full((L,), jnp.uint32(0xFFFFFFFE))))
            bits = jnp.where(lo >= jnp.full((L,), jnp.uint32(0x80000000)),
                             lo ^ jnp.uint32(0x80000000), ~lo)
            tv = plsc.bitcast(bits, jnp.float32)
            tvr[...] = tv

            def comp_step(j, c2):
                v = cv[pl.ds(j * L, L)]
                ix = ci[pl.ds(j * L, L)]
                keep = v >= tv
                n = _popcount(keep)

                @pl.when(n > 0)
                def _():
                    plsc.store_compressed(cv.at[pl.ds(c2, L)], v, mask=keep)
                    plsc.store_compressed(ci.at[pl.ds(c2, L)], ix, mask=keep)

                return c2 + n

            c2 = lax.fori_loop(0, NBUF, comp_step, jnp.int32(0))
            smc[0] = c2

            def pad_step(j, carry):
                v = cv[pl.ds(j * L, L)]
                keep_lane = (lane + j * L) < c2
                cv[pl.ds(j * L, L)] = jnp.where(keep_lane, v, jnp.float32(NEG_INF))
                return carry

            lax.fori_loop(0, NBUF, pad_step, 0)

        def boot_step(i, carry):
            cv[pl.ds(i * L, L)] = row_v[pl.ds(i * L, L)]
            ci[pl.ds(i * L, L)] = lane + i * L
            return carry

        def group_step(gi, carry):
            tv = tvr[...]
            base = (BOOT + gi * G) * L
            mx = row_v[pl.ds(base, L)]
            for k in range(1, G):
                mx = jnp.maximum(mx, row_v[pl.ds(base + k * L, L)])
            gn = _popcount(mx >= tv)

            @pl.when(gn > 0)
            def _():
                vs = [row_v[pl.ds(base + k * L, L)] for k in range(G)]
                masks = [v >= tv for v in vs]
                ns = [_popcount(m) for m in masks]
                for k in range(G):
                    c = smc[0]

                    @pl.when(ns[k] > 0)
                    def _(k=k, c=c):
                        plsc.store_compressed(cv.at[pl.ds(c, L)], vs[k],
                                              mask=masks[k])
                        plsc.store_compressed(ci.at[pl.ds(c, L)],
                                              lane + base + k * L,
                                              mask=masks[k])
                        smc[0] = c + ns[k]

                @pl.when(smc[0] > RESEL_AT)
                def _():
                    reselect(32, 20)

            return carry

        for rr in range(rows_per_w):
            r = wid * rows_per_w + rr
            pltpu.sync_copy(logits_hbm.at[r], row_v)
            lax.fori_loop(0, BOOT, boot_step, 0)
            cv[pl.ds(BOOT * L, L)] = jnp.full((L,), jnp.float32(NEG_INF))
            smc[0] = jnp.int32(BOOT * L)
            reselect(32, 20)
            lax.fori_loop(0, ngroups, group_step, 0)

            @pl.when(smc[0] > OUTCAP)
            def _():
                reselect(NBUF, 32)

            pltpu.sync_copy(cv.at[pl.ds(0, OUTCAP)], vals_hbm.at[r])
            pltpu.sync_copy(ci.at[pl.ds(0, OUTCAP)], idx_hbm.at[r])

    return body(logits)


def _tail_body(kd_ref, vals_ref, idx_ref, out_ref, *, vocab):
    v = vals_ref[...]
    ix = idx_ref[...]
    B = v.shape[0]
    k0 = kd_ref[0]
    k1 = kd_ref[1]

    sb = lax.bitcast_convert_type(v, jnp.int32)
    neg = lax.bitcast_convert_type(sb >> 31, jnp.uint32)
    keys = lax.bitcast_convert_type(sb, jnp.uint32) ^ (
        neg | jnp.uint32(0x80000000))

    lo = jnp.zeros((B, 1), jnp.uint32)
    hi = jnp.full((B, 1), jnp.uint32(0xFFFFFFFE))
    for _ in range(32):
        mid = lo + ((hi - lo + jnp.uint32(1)) >> jnp.uint32(1))
        cnt = jnp.sum((keys >= mid).astype(jnp.int32), axis=1, keepdims=True)
        big = cnt >= TOPK
        lo = jnp.where(big, mid, lo)
        hi = jnp.where(big, hi, mid - jnp.uint32(1))
    mk = lo
    mv_bits = jnp.where(mk >= jnp.uint32(0x80000000),
                        mk ^ jnp.uint32(0x80000000), ~mk)
    min_val = lax.bitcast_convert_type(mv_bits, jnp.float32)

    m = jnp.max(v, axis=1, keepdims=True)
    sup = v >= min_val
    e = jnp.where(sup, jnp.exp(v - m), jnp.float32(0.0))
    denom = jnp.sum(e, axis=1, keepdims=True)
    lp = jnp.log(e / denom + jnp.float32(1e-30))

    row = lax.broadcasted_iota(jnp.int32, v.shape, 0)
    lin = (row * vocab + ix).astype(jnp.uint32)
    x0 = jnp.zeros_like(lin)
    x1 = lin
    ks2 = k0 ^ k1 ^ jnp.uint32(0x1BD11BDA)

    def rotl(x, r):
        return (x << jnp.uint32(r)) | (x >> jnp.uint32(32 - r))

    x0 = x0 + k0
    x1 = x1 + k1
    rots = ((13, 15, 26, 6), (17, 29, 16, 24))
    kadd = ((k1, ks2 + jnp.uint32(1)), (ks2, k0 + jnp.uint32(2)),
            (k0, k1 + jnp.uint32(3)), (k1, ks2 + jnp.uint32(4)),
            (ks2, k0 + jnp.uint32(5)))
    for i in range(5):
        for r in rots[i % 2]:
            x0 = x0 + x1
            x1 = rotl(x1, r)
            x1 = x1 ^ x0
        x0 = x0 + kadd[i][0]
        x1 = x1 + kadd[i][1]
    bits = x0 ^ x1

    fb = (bits >> jnp.uint32(9)) | jnp.uint32(0x3F800000)
    u = lax.bitcast_convert_type(fb, jnp.float32) - jnp.float32(1.0)
    tiny = np.float32(np.finfo(np.float32).tiny)
    u2 = jnp.maximum(tiny, u * (np.float32(1.0) - tiny) + tiny)
    g = -jnp.log(-jnp.log(u2))

    score = jnp.where(sup, lp + g, -jnp.inf)
    best = jnp.max(score, axis=1, keepdims=True)
    samp = jnp.min(jnp.where(score == best, ix, jnp.int32(0x7FFFFFFF)),
                   axis=1, keepdims=True)
    out_ref[...] = samp


def _tail(key_data, vals, idx, vocab):
    B = vals.shape[0]
    return pl.pallas_call(
        functools.partial(_tail_body, vocab=vocab),
        out_shape=jax.ShapeDtypeStruct((B, 1), jnp.int32),
        in_specs=[
            pl.BlockSpec(memory_space=pltpu.SMEM),
            pl.BlockSpec(memory_space=pltpu.VMEM),
            pl.BlockSpec(memory_space=pltpu.VMEM),
        ],
        out_specs=pl.BlockSpec(memory_space=pltpu.VMEM),
    )(key_data, vals, idx)


def kernel(logits):
    if logits.ndim == 3:
        logits = jnp.squeeze(logits, axis=1)
    B, V = logits.shape
    vals, idx = _sc_collect(logits)
    kd = jax.random.key_data(
        jax.random.fold_in(jax.random.key(0), 1)).astype(jnp.uint32)
    return _tail(kd, vals, idx, V)
